# baseline (speedup 1.0000x reference)
import functools

import jax
import jax.numpy as jnp
from jax.experimental import pallas as pl
from jax.experimental.pallas import tpu as pltpu

PI = 3.141592653589793
EPS = 1e-7

B, A, NC = 16, 21504, 80
N = B * A
CHUNK = 2048
GRID = N // CHUNK


def _acos(x):
    ax = jnp.abs(x)
    p = (1.5707963050 + ax * (-0.2145988016 + ax * (0.0889789874 + ax * (
        -0.0501743046 + ax * (0.0308918810 + ax * (-0.0170881256 + ax * (
            0.0066700901 + ax * -0.0012624911)))))))
    r = jnp.sqrt(jnp.maximum(1.0 - ax, 0.0)) * p
    return jnp.where(x >= 0.0, r, PI - r)


def _loss_body(scores_ref, pb_ref, tb_ref, m_ref, iou_out, dist_out):
    i = pl.program_id(0)

    s = scores_ref[...]
    w = jnp.sum(s, axis=1, keepdims=True)

    p = pb_ref[...]
    t = tb_ref[...]
    m = m_ref[...]

    x1 = p[:, 0:1]
    y1 = p[:, 1:2]
    r1 = p[:, 2:3]
    x2 = t[:, 0:1]
    y2 = t[:, 1:2]
    r2 = t[:, 2:3]

    d2 = (x1 - x2) ** 2 + (y1 - y2) ** 2
    d = jnp.sqrt(jnp.maximum(d2, EPS))
    rsum = r1 + r2
    rdiff = jnp.abs(r1 - r2)
    rmin = jnp.minimum(r1, r2)
    no_overlap = d >= rsum
    contained = d <= rdiff
    a1 = jnp.clip((d2 + r1 ** 2 - r2 ** 2) / (2.0 * d * jnp.maximum(r1, EPS)),
                  -1.0 + 1e-6, 1.0 - 1e-6)
    a2 = jnp.clip((d2 + r2 ** 2 - r1 ** 2) / (2.0 * d * jnp.maximum(r2, EPS)),
                  -1.0 + 1e-6, 1.0 - 1e-6)
    tri = jnp.maximum((-d + rsum) * (d + r1 - r2) * (d - r1 + r2) * (d + rsum),
                      EPS)
    lens = (r1 ** 2 * _acos(a1) + r2 ** 2 * _acos(a2)
            - 0.5 * jnp.sqrt(tri))
    inter = jnp.where(no_overlap, 0.0, jnp.where(contained, PI * rmin ** 2, lens))
    union = PI * (r1 ** 2 + r2 ** 2) - inter
    iou = inter / (union + EPS)

    dist = jnp.clip(1.0 - d / (rsum + EPS), 0.0, 1.0)

    wm = w * m
    part_iou = jnp.sum((1.0 - iou) * wm)
    part_dist = jnp.sum((1.0 - dist) * wm)

    @pl.when(i == 0)
    def _init():
        iou_out[0, 0] = 0.0
        dist_out[0, 0] = 0.0

    iou_out[0, 0] += part_iou
    dist_out[0, 0] += part_dist


@jax.jit
def _loss_sums(scores, pb, tb, m):
    return pl.pallas_call(
        _loss_body,
        grid=(GRID,),
        in_specs=[
            pl.BlockSpec((CHUNK, NC), lambda i: (i, 0)),
            pl.BlockSpec((CHUNK, 3), lambda i: (i, 0)),
            pl.BlockSpec((CHUNK, 3), lambda i: (i, 0)),
            pl.BlockSpec((CHUNK, 1), lambda i: (i, 0)),
        ],
        out_specs=[
            pl.BlockSpec(memory_space=pltpu.SMEM),
            pl.BlockSpec(memory_space=pltpu.SMEM),
        ],
        out_shape=[
            jax.ShapeDtypeStruct((1, 1), jnp.float32),
            jax.ShapeDtypeStruct((1, 1), jnp.float32),
        ],
    )(scores, pb, tb, m)


def kernel(pred_dist, pred_bboxes, anchor_points, target_bboxes,
           target_scores, target_scores_sum, fg_mask):
    scores = target_scores.reshape(N, NC)
    pb = pred_bboxes.reshape(N, 3)
    tb = target_bboxes.reshape(N, 3)
    m = fg_mask.reshape(N, 1).astype(jnp.float32)
    si, sd = _loss_sums(scores, pb, tb, m)
    inv = 1.0 / target_scores_sum
    return (si[0, 0] * inv, sd[0, 0] * inv)


# speedup vs baseline: 2.1963x; 2.1963x over previous
import jax
import jax.numpy as jnp
from jax import lax
from jax.experimental import pallas as pl
from jax.experimental.pallas import tpu as pltpu

PI = 3.141592653589793
EPS = 1e-7

B, A, NC = 16, 21504, 80
N = B * A
TILE = 128 * 128
GRID = N // TILE
KDIM = 128 * NC


def _acos(x):
    ax = jnp.abs(x)
    p = (1.5707963050 + ax * (-0.2145988016 + ax * (0.0889789874 + ax * (
        -0.0501743046 + ax * (0.0308918810 + ax * (-0.0170881256 + ax * (
            0.0066700901 + ax * -0.0012624911)))))))
    r = jnp.sqrt(jnp.maximum(1.0 - ax, 0.0)) * p
    return jnp.where(x >= 0.0, r, PI - r)


def _loss_body(s_ref, pb_ref, tb_ref, m_ref, iou_out, dist_out,
               k80_ref, kc_ref):
    i = pl.program_id(0)

    @pl.when(i == 0)
    def _init():
        row = lax.broadcasted_iota(jnp.int32, (KDIM, 128), 0)
        col = lax.broadcasted_iota(jnp.int32, (KDIM, 128), 1)
        k80_ref[...] = (row // NC == col).astype(jnp.float32)
        rr = lax.broadcasted_iota(jnp.int32, (384, 384), 0)
        cc = lax.broadcasted_iota(jnp.int32, (384, 384), 1)
        kc_ref[...] = (rr == 3 * (cc % 128) + cc // 128).astype(jnp.float32)
        iou_out[0, 0] = 0.0
        dist_out[0, 0] = 0.0

    w = lax.dot(s_ref[...], k80_ref[...],
                precision=lax.Precision.DEFAULT)

    pxyz = lax.dot(pb_ref[...], kc_ref[...],
                   precision=lax.Precision.HIGHEST)
    txyz = lax.dot(tb_ref[...], kc_ref[...],
                   precision=lax.Precision.HIGHEST)
    x1 = pxyz[:, 0:128]
    y1 = pxyz[:, 128:256]
    r1 = pxyz[:, 256:384]
    x2 = txyz[:, 0:128]
    y2 = txyz[:, 128:256]
    r2 = txyz[:, 256:384]
    m = m_ref[...].astype(jnp.float32)

    d2 = (x1 - x2) ** 2 + (y1 - y2) ** 2
    d = jnp.sqrt(jnp.maximum(d2, EPS))
    rsum = r1 + r2
    rdiff = jnp.abs(r1 - r2)
    rmin = jnp.minimum(r1, r2)
    no_overlap = d >= rsum
    contained = d <= rdiff
    a1 = jnp.clip((d2 + r1 ** 2 - r2 ** 2) / (2.0 * d * jnp.maximum(r1, EPS)),
                  -1.0 + 1e-6, 1.0 - 1e-6)
    a2 = jnp.clip((d2 + r2 ** 2 - r1 ** 2) / (2.0 * d * jnp.maximum(r2, EPS)),
                  -1.0 + 1e-6, 1.0 - 1e-6)
    tri = jnp.maximum((-d + rsum) * (d + r1 - r2) * (d - r1 + r2) * (d + rsum),
                      EPS)
    lens = (r1 ** 2 * _acos(a1) + r2 ** 2 * _acos(a2)
            - 0.5 * jnp.sqrt(tri))
    inter = jnp.where(no_overlap, 0.0, jnp.where(contained, PI * rmin ** 2, lens))
    union = PI * (r1 ** 2 + r2 ** 2) - inter
    iou = inter / (union + EPS)

    dist = jnp.clip(1.0 - d / (rsum + EPS), 0.0, 1.0)

    wm = w * m
    iou_out[0, 0] += jnp.sum((1.0 - iou) * wm)
    dist_out[0, 0] += jnp.sum((1.0 - dist) * wm)


@jax.jit
def _loss_sums(scores, pb, tb, m):
    return pl.pallas_call(
        _loss_body,
        grid=(GRID,),
        in_specs=[
            pl.BlockSpec((128, KDIM), lambda i: (i, 0)),
            pl.BlockSpec((128, 384), lambda i: (i, 0)),
            pl.BlockSpec((128, 384), lambda i: (i, 0)),
            pl.BlockSpec((128, 128), lambda i: (i, 0)),
        ],
        out_specs=[
            pl.BlockSpec(memory_space=pltpu.SMEM),
            pl.BlockSpec(memory_space=pltpu.SMEM),
        ],
        out_shape=[
            jax.ShapeDtypeStruct((1, 1), jnp.float32),
            jax.ShapeDtypeStruct((1, 1), jnp.float32),
        ],
        scratch_shapes=[
            pltpu.VMEM((KDIM, 128), jnp.float32),
            pltpu.VMEM((384, 384), jnp.float32),
        ],
    )(scores, pb, tb, m)


def kernel(pred_dist, pred_bboxes, anchor_points, target_bboxes,
           target_scores, target_scores_sum, fg_mask):
    scores = target_scores.reshape(GRID, 128, KDIM).reshape(GRID * 128, KDIM)
    pb = pred_bboxes.reshape(GRID * 128, 384)
    tb = target_bboxes.reshape(GRID * 128, 384)
    m = fg_mask.reshape(GRID * 128, 128)
    si, sd = _loss_sums(scores, pb, tb, m)
    inv = 1.0 / target_scores_sum
    return (si[0, 0] * inv, sd[0, 0] * inv)


# speedup vs baseline: 25.3448x; 11.5399x over previous
import jax
import jax.numpy as jnp
from jax import lax
from jax.experimental import pallas as pl
from jax.experimental.pallas import tpu as pltpu

PI = 3.141592653589793
EPS = 1e-7

B, A, NC = 16, 21504, 80
ABLK = 1024
GRID = A // ABLK


def _acos(x):
    ax = jnp.abs(x)
    p = (1.5707963050 + ax * (-0.2145988016 + ax * (0.0889789874 + ax * (
        -0.0501743046 + ax * (0.0308918810 + ax * (-0.0170881256 + ax * (
            0.0066700901 + ax * -0.0012624911)))))))
    r = jnp.sqrt(jnp.maximum(1.0 - ax, 0.0)) * p
    return jnp.where(x >= 0.0, r, PI - r)


def _loss_body(s_ref, p_ref, t_ref, m_ref, iou_out, dist_out):
    i = pl.program_id(0)

    @pl.when(i == 0)
    def _init():
        iou_out[0, 0] = 0.0
        dist_out[0, 0] = 0.0

    w = jnp.sum(s_ref[...], axis=1)
    x1 = p_ref[:, 0, :]
    y1 = p_ref[:, 1, :]
    r1 = p_ref[:, 2, :]
    x2 = t_ref[:, 0, :]
    y2 = t_ref[:, 1, :]
    r2 = t_ref[:, 2, :]
    m = m_ref[...]

    d2 = (x1 - x2) ** 2 + (y1 - y2) ** 2
    d = jnp.sqrt(jnp.maximum(d2, EPS))
    rsum = r1 + r2
    rdiff = jnp.abs(r1 - r2)
    rmin = jnp.minimum(r1, r2)
    no_overlap = d >= rsum
    contained = d <= rdiff
    a1 = jnp.clip((d2 + r1 ** 2 - r2 ** 2) / (2.0 * d * jnp.maximum(r1, EPS)),
                  -1.0 + 1e-6, 1.0 - 1e-6)
    a2 = jnp.clip((d2 + r2 ** 2 - r1 ** 2) / (2.0 * d * jnp.maximum(r2, EPS)),
                  -1.0 + 1e-6, 1.0 - 1e-6)
    tri = jnp.maximum((-d + rsum) * (d + r1 - r2) * (d - r1 + r2) * (d + rsum),
                      EPS)
    lens = (r1 ** 2 * _acos(a1) + r2 ** 2 * _acos(a2)
            - 0.5 * jnp.sqrt(tri))
    inter = jnp.where(no_overlap, 0.0, jnp.where(contained, PI * rmin ** 2, lens))
    union = PI * (r1 ** 2 + r2 ** 2) - inter
    iou = inter / (union + EPS)

    dist = jnp.clip(1.0 - d / (rsum + EPS), 0.0, 1.0)

    wm = w * m
    iou_out[0, 0] += jnp.sum((1.0 - iou) * wm)
    dist_out[0, 0] += jnp.sum((1.0 - dist) * wm)


@jax.jit
def _loss_sums(st, pt, tt, mt):
    return pl.pallas_call(
        _loss_body,
        grid=(GRID,),
        in_specs=[
            pl.BlockSpec((B, NC, ABLK), lambda i: (0, 0, i)),
            pl.BlockSpec((B, 3, ABLK), lambda i: (0, 0, i)),
            pl.BlockSpec((B, 3, ABLK), lambda i: (0, 0, i)),
            pl.BlockSpec((B, ABLK), lambda i: (0, i)),
        ],
        out_specs=[
            pl.BlockSpec(memory_space=pltpu.SMEM),
            pl.BlockSpec(memory_space=pltpu.SMEM),
        ],
        out_shape=[
            jax.ShapeDtypeStruct((1, 1), jnp.float32),
            jax.ShapeDtypeStruct((1, 1), jnp.float32),
        ],
    )(st, pt, tt, mt)


def kernel(pred_dist, pred_bboxes, anchor_points, target_bboxes,
           target_scores, target_scores_sum, fg_mask):
    st = jnp.transpose(target_scores, (0, 2, 1))
    pt = jnp.transpose(pred_bboxes, (0, 2, 1))
    tt = jnp.transpose(target_bboxes, (0, 2, 1))
    mt = fg_mask.astype(jnp.float32)
    si, sd = _loss_sums(st, pt, tt, mt)
    inv = 1.0 / target_scores_sum
    return (si[0, 0] * inv, sd[0, 0] * inv)


# speedup vs baseline: 26.3587x; 1.0400x over previous
import jax
import jax.numpy as jnp
from jax import lax
from jax.experimental import pallas as pl
from jax.experimental.pallas import tpu as pltpu

PI = 3.141592653589793
EPS = 1e-7

B, A, NC = 16, 21504, 80
ABLK = 2688
GRID = A // ABLK


def _acos(x):
    ax = jnp.abs(x)
    p = (1.5707963050 + ax * (-0.2145988016 + ax * (0.0889789874 + ax * (
        -0.0501743046 + ax * (0.0308918810 + ax * (-0.0170881256 + ax * (
            0.0066700901 + ax * -0.0012624911)))))))
    r = jnp.sqrt(jnp.maximum(1.0 - ax, 0.0)) * p
    return jnp.where(x >= 0.0, r, PI - r)


def _loss_body(s_ref, p_ref, t_ref, m_ref, iou_out, dist_out):
    i = pl.program_id(0)

    @pl.when(i == 0)
    def _init():
        iou_out[0, 0] = 0.0
        dist_out[0, 0] = 0.0

    w = jnp.sum(s_ref[...], axis=1)
    x1 = p_ref[:, 0, :]
    y1 = p_ref[:, 1, :]
    r1 = p_ref[:, 2, :]
    x2 = t_ref[:, 0, :]
    y2 = t_ref[:, 1, :]
    r2 = t_ref[:, 2, :]
    m = m_ref[...]

    d2 = (x1 - x2) ** 2 + (y1 - y2) ** 2
    d = jnp.sqrt(jnp.maximum(d2, EPS))
    rsum = r1 + r2
    rdiff = jnp.abs(r1 - r2)
    rmin = jnp.minimum(r1, r2)
    no_overlap = d >= rsum
    contained = d <= rdiff
    a1 = jnp.clip((d2 + r1 ** 2 - r2 ** 2) / (2.0 * d * jnp.maximum(r1, EPS)),
                  -1.0 + 1e-6, 1.0 - 1e-6)
    a2 = jnp.clip((d2 + r2 ** 2 - r1 ** 2) / (2.0 * d * jnp.maximum(r2, EPS)),
                  -1.0 + 1e-6, 1.0 - 1e-6)
    tri = jnp.maximum((-d + rsum) * (d + r1 - r2) * (d - r1 + r2) * (d + rsum),
                      EPS)
    lens = (r1 ** 2 * _acos(a1) + r2 ** 2 * _acos(a2)
            - 0.5 * jnp.sqrt(tri))
    inter = jnp.where(no_overlap, 0.0, jnp.where(contained, PI * rmin ** 2, lens))
    union = PI * (r1 ** 2 + r2 ** 2) - inter
    iou = inter / (union + EPS)

    dist = jnp.clip(1.0 - d / (rsum + EPS), 0.0, 1.0)

    wm = w * m
    iou_out[0, 0] += jnp.sum((1.0 - iou) * wm)
    dist_out[0, 0] += jnp.sum((1.0 - dist) * wm)


@jax.jit
def _loss_sums(st, pt, tt, mt):
    return pl.pallas_call(
        _loss_body,
        grid=(GRID,),
        in_specs=[
            pl.BlockSpec((B, NC, ABLK), lambda i: (0, 0, i)),
            pl.BlockSpec((B, 3, ABLK), lambda i: (0, 0, i)),
            pl.BlockSpec((B, 3, ABLK), lambda i: (0, 0, i)),
            pl.BlockSpec((B, ABLK), lambda i: (0, i)),
        ],
        out_specs=[
            pl.BlockSpec(memory_space=pltpu.SMEM),
            pl.BlockSpec(memory_space=pltpu.SMEM),
        ],
        out_shape=[
            jax.ShapeDtypeStruct((1, 1), jnp.float32),
            jax.ShapeDtypeStruct((1, 1), jnp.float32),
        ],
    )(st, pt, tt, mt)


def kernel(pred_dist, pred_bboxes, anchor_points, target_bboxes,
           target_scores, target_scores_sum, fg_mask):
    st = jnp.transpose(target_scores, (0, 2, 1))
    pt = jnp.transpose(pred_bboxes, (0, 2, 1))
    tt = jnp.transpose(target_bboxes, (0, 2, 1))
    mt = fg_mask.astype(jnp.float32)
    si, sd = _loss_sums(st, pt, tt, mt)
    inv = 1.0 / target_scores_sum
    return (si[0, 0] * inv, sd[0, 0] * inv)
